# baseline (speedup 1.0000x reference)
import functools

import numpy as np
import jax
import jax.numpy as jnp
from jax import lax
from jax.experimental import pallas as pl
from jax.experimental.pallas import tpu as pltpu
from jax.experimental.pallas import tpu_sc as plsc

H_IMG = 800.0
W_IMG = 1333.0
NMS_THR = 0.7
MIN_FINAL = 5
MAX_FINAL = 50
NUM_IMG = 4
NUM_INIT = 5000

_NPAD = 5120
_SEG = _NPAD // 8
_SLOT = 64
_ROUNDS = MAX_FINAL - 1
_BIGF = np.float32(2 ** 24)
_H_MIN = np.float32(H_IMG * 0.1)
_W_MIN = np.float32(W_IMG * 0.1)


def _sc_nms(a_hbm, b_hbm, c_hbm, d_hbm, ps_hbm, nb_hbm,
            ox1_hbm, oy1_hbm, ox2_hbm, oy2_hbm, cnt_hbm,
            x1_v, y1_v, x2_v, y2_v, sc_v, nb_v,
            ox1_v, oy1_v, ox2_v, oy2_v, cnt_v,
            cand_v, grp_v, sib_v, ar_v, cand_sp):
    ci = lax.axis_index("c")
    si = lax.axis_index("s")
    grp = lax.shift_right_logical(si, 3)
    img = ci * 2 + grp
    slot = si & 7
    seg = slot * _SEG

    pltpu.sync_copy(a_hbm.at[img, pl.ds(seg, _SEG)], x1_v)
    pltpu.sync_copy(b_hbm.at[img, pl.ds(seg, _SEG)], y1_v)
    pltpu.sync_copy(c_hbm.at[img, pl.ds(seg, _SEG)], x2_v)
    pltpu.sync_copy(d_hbm.at[img, pl.ds(seg, _SEG)], y2_v)
    pltpu.sync_copy(ps_hbm.at[img, pl.ds(seg, _SEG)], sc_v)
    pltpu.sync_copy(nb_hbm, nb_v)

    lane = lax.iota(jnp.int32, 16)

    @plsc.parallel_loop(
        0, _SEG, step=16, unroll=8,
        carry=(jnp.full((16,), -2.0, jnp.float32),
               jnp.zeros((16,), jnp.int32)))
    def prep_carry(i, carry):
        rm, rc = carry
        s = pl.ds(i, 16)
        av = x1_v[s]
        bv = y1_v[s]
        cv = x2_v[s]
        dv = y2_v[s]
        x1 = jnp.minimum(av, cv) * W_IMG
        x2 = jnp.maximum(av, cv) * W_IMG
        y1 = jnp.minimum(bv, dv) * H_IMG
        y2 = jnp.maximum(bv, dv) * H_IMG
        bw = x2 - x1
        bh = y2 - y1
        colv = seg + i + lane
        m = (bh > _H_MIN) & (bw > _W_MIN) & (colv < NUM_INIT)
        sc = jnp.where(m, sc_v[s], -1.0)
        x1_v[s] = x1
        y1_v[s] = y1
        x2_v[s] = x2
        y2_v[s] = y2
        sc_v[s] = sc
        ar_v[s] = bw * bh
        upd = sc > rm
        return jnp.maximum(rm, sc), jnp.where(upd, i, rc)

    rm0, rc0 = prep_carry

    zf = jnp.zeros((16,), jnp.float32)
    for j in range(_SLOT // 16):
        s = pl.ds(j * 16, 16)
        ox1_v[s] = zf
        oy1_v[s] = zf
        ox2_v[s] = zf
        oy2_v[s] = zf

    nbv = nb_v[...].astype(jnp.float32)
    nf_f = jnp.sum(jnp.where(lane == img, nbv, 0.0))
    nf = jnp.clip(nf_f, np.float32(MIN_FINAL),
                  np.float32(MAX_FINAL - 1)).astype(jnp.int32)

    def round_cond(st):
        return st[0]

    def round_body(st):
        cont, r, k, act, rm, rc = st
        lmax = jnp.max(rm)
        lmax_v = jnp.full((16,), lmax, jnp.float32)
        gi_f = jnp.min(jnp.where(rm == lmax_v,
                                 (rc + lane + seg).astype(jnp.float32),
                                 _BIGF))
        gl = jnp.minimum(gi_f.astype(jnp.int32) - seg, _SEG - 1)
        gbase = lax.shift_left(lax.shift_right_logical(gl, 4), 4)
        gsel = lane == (gl & 15)
        gs = pl.ds(gbase, 16)
        x1l = jnp.full((16,), jnp.sum(jnp.where(gsel, x1_v[gs], 0.0)),
                       jnp.float32)
        y1l = jnp.full((16,), jnp.sum(jnp.where(gsel, y1_v[gs], 0.0)),
                       jnp.float32)
        x2l = jnp.full((16,), jnp.sum(jnp.where(gsel, x2_v[gs], 0.0)),
                       jnp.float32)
        y2l = jnp.full((16,), jnp.sum(jnp.where(gsel, y2_v[gs], 0.0)),
                       jnp.float32)
        cand_v[pl.ds(0, 16)] = lmax_v
        cand_v[pl.ds(16, 16)] = jnp.full((16,), gi_f, jnp.float32)
        cand_v[pl.ds(32, 16)] = x1l
        cand_v[pl.ds(48, 16)] = y1l
        cand_v[pl.ds(64, 16)] = x2l
        cand_v[pl.ds(80, 16)] = y2l
        cand_v[pl.ds(96, 16)] = jnp.where(act, jnp.full((16,), 1.0),
                                          jnp.full((16,), 0.0))

        par = r & 1
        woff = pl.multiple_of(par * 1792 + si * 112, 112)
        pltpu.sync_copy(cand_v, cand_sp.at[pl.ds(woff, 112)])
        plsc.subcore_barrier()
        roff = pl.multiple_of(par * 1792 + grp * 896, 896)
        pltpu.sync_copy(cand_sp.at[pl.ds(roff, 896)], grp_v)
        soff = pl.multiple_of(par * 1792 + (1 - grp) * 896 + 96, 16)
        pltpu.sync_copy(cand_sp.at[pl.ds(soff, 16)], sib_v)

        gb = 0
        sj = [grp_v[pl.ds(gb + j * 112, 16)] for j in range(8)]
        ij = [grp_v[pl.ds(gb + j * 112 + 16, 16)] for j in range(8)]
        wmax = sj[0]
        for j in range(1, 8):
            wmax = jnp.maximum(wmax, sj[j])
        widx = _BIGF * jnp.ones((16,), jnp.float32)
        for j in range(8):
            widx = jnp.minimum(widx, jnp.where(sj[j] == wmax, ij[j], _BIGF))
        selj = [(sj[j] == wmax) & (ij[j] == widx) for j in range(8)]

        def pick(off):
            acc = jnp.zeros((16,), jnp.float32)
            for j in range(8):
                acc = acc + jnp.where(selj[j],
                                      grp_v[pl.ds(gb + j * 112 + off, 16)],
                                      0.0)
            return acc

        x1m = pick(32)
        y1m = pick(48)
        x2m = pick(64)
        y2m = pick(80)
        am = (x2m - x1m) * (y2m - y1m)
        found = jnp.max(wmax) > -0.5

        @plsc.parallel_loop(
            0, _SEG, step=16, unroll=8,
            carry=(jnp.full((16,), -2.0, jnp.float32),
                   jnp.zeros((16,), jnp.int32)))
        def supp_carry(i, carry):
            rm2, rc2 = carry
            s = pl.ds(i, 16)
            x1 = x1_v[s]
            y1 = y1_v[s]
            x2 = x2_v[s]
            y2 = y2_v[s]
            ar = ar_v[s]
            sc = sc_v[s]
            xx1 = jnp.maximum(x1m, x1)
            yy1 = jnp.maximum(y1m, y1)
            xx2 = jnp.minimum(x2m, x2)
            yy2 = jnp.minimum(y2m, y2)
            w = jnp.maximum(0.0, xx2 - xx1)
            h = jnp.maximum(0.0, yy2 - yy1)
            inter = w * h
            iou = inter / (am + ar - inter + 1e-9)
            sc2 = jnp.where(iou > NMS_THR, -1.0, sc)
            sc_v[s] = sc2
            upd = sc2 > rm2
            return jnp.maximum(rm2, sc2), jnp.where(upd, i, rc2)

        rm3, rc3 = supp_carry

        write = found & (k < nf)

        @pl.when((slot == 0) & write)
        def _():
            kbase = lax.shift_left(lax.shift_right_logical(k, 4), 4)
            ks = pl.ds(kbase, 16)
            wsel = lane == (k & 15)
            ox1_v[ks] = jnp.where(wsel, x1m, ox1_v[ks])
            oy1_v[ks] = jnp.where(wsel, y1m, oy1_v[ks])
            ox2_v[ks] = jnp.where(wsel, x2m, ox2_v[ks])
            oy2_v[ks] = jnp.where(wsel, y2m, oy2_v[ks])

        k2 = k + jnp.where(write, 1, 0).astype(jnp.int32)
        act2 = write & (k2 < nf)

        sib = jnp.max(sib_v[...]) > 0.5
        cont2 = act | sib
        return cont2, r + 1, k2, act2, rm3, rc3

    _, _, k_fin, _, _, _ = lax.while_loop(
        round_cond, round_body,
        (jnp.bool_(True), jnp.int32(0), jnp.int32(0), jnp.bool_(True),
         rm0, rc0))

    @pl.when(slot == 0)
    def _():
        pltpu.sync_copy(ox1_v, ox1_hbm.at[img])
        pltpu.sync_copy(oy1_v, oy1_hbm.at[img])
        pltpu.sync_copy(ox2_v, ox2_hbm.at[img])
        pltpu.sync_copy(oy2_v, oy2_hbm.at[img])
        cnt_v[...] = jnp.full((16,), k_fin, jnp.int32)
        pltpu.sync_copy(cnt_v, cnt_hbm.at[img])


@functools.lru_cache(maxsize=1)
def _build_sc_kernel():
    mesh = plsc.VectorSubcoreMesh(core_axis_name="c", subcore_axis_name="s")
    f_out = jax.ShapeDtypeStruct((NUM_IMG, _SLOT), jnp.float32)
    i_out = jax.ShapeDtypeStruct((NUM_IMG, 16), jnp.int32)
    seg = pltpu.VMEM((_SEG,), jnp.float32)
    return pl.kernel(
        _sc_nms,
        out_type=(f_out, f_out, f_out, f_out, i_out),
        mesh=mesh,
        compiler_params=pltpu.CompilerParams(needs_layout_passes=False),
        scratch_types=[
            seg, seg, seg, seg, seg,
            pltpu.VMEM((16,), jnp.int32),
            pltpu.VMEM((_SLOT,), jnp.float32),
            pltpu.VMEM((_SLOT,), jnp.float32),
            pltpu.VMEM((_SLOT,), jnp.float32),
            pltpu.VMEM((_SLOT,), jnp.float32),
            pltpu.VMEM((16,), jnp.int32),
            pltpu.VMEM((112,), jnp.float32),
            pltpu.VMEM((896,), jnp.float32),
            pltpu.VMEM((16,), jnp.float32),
            seg,
            pltpu.VMEM_SHARED((3584,), jnp.float32),
        ],
    )


def kernel(rand_boxes_init, pseudo_scores, num_of_boxes_per_img):
    pad = _NPAD - NUM_INIT
    a = jnp.pad(rand_boxes_init[..., 0], ((0, 0), (0, pad)))
    b = jnp.pad(rand_boxes_init[..., 1], ((0, 0), (0, pad)))
    c = jnp.pad(rand_boxes_init[..., 2], ((0, 0), (0, pad)))
    d = jnp.pad(rand_boxes_init[..., 3], ((0, 0), (0, pad)))
    ps = jnp.pad(pseudo_scores, ((0, 0), (0, pad)))
    nb = jnp.pad(num_of_boxes_per_img, (0, 16 - NUM_IMG))

    ox1, oy1, ox2, oy2, cnt = _build_sc_kernel()(a, b, c, d, ps, nb)

    out = jnp.stack([ox1[:, :MAX_FINAL], oy1[:, :MAX_FINAL],
                     ox2[:, :MAX_FINAL], oy2[:, :MAX_FINAL]], axis=-1)
    counts = cnt[:, 0]
    return out, counts


# speedup vs baseline: 1.0414x; 1.0414x over previous
import functools

import numpy as np
import jax
import jax.numpy as jnp
from jax import lax
from jax.experimental import pallas as pl
from jax.experimental.pallas import tpu as pltpu
from jax.experimental.pallas import tpu_sc as plsc

H_IMG = 800.0
W_IMG = 1333.0
NMS_THR = 0.7
MIN_FINAL = 5
MAX_FINAL = 50
NUM_IMG = 4
NUM_INIT = 5000

_NPAD = 5120
_SEG = _NPAD // 8
_SLOT = 64
_ROUNDS = MAX_FINAL - 1
_BIGF = np.float32(2 ** 24)
_H_MIN = np.float32(H_IMG * 0.1)
_W_MIN = np.float32(W_IMG * 0.1)


def _sc_nms(a_hbm, b_hbm, c_hbm, d_hbm, ps_hbm, nb_hbm,
            ox1_hbm, oy1_hbm, ox2_hbm, oy2_hbm, cnt_hbm,
            x1_v, y1_v, x2_v, y2_v, sc_v, nb_v,
            ox1_v, oy1_v, ox2_v, oy2_v, cnt_v,
            cand_v, grp_v, ar_v, cand_sp):
    ci = lax.axis_index("c")
    si = lax.axis_index("s")
    grp = lax.shift_right_logical(si, 3)
    img = ci * 2 + grp
    slot = si & 7
    seg = slot * _SEG

    pltpu.sync_copy(a_hbm.at[img, pl.ds(seg, _SEG)], x1_v)
    pltpu.sync_copy(b_hbm.at[img, pl.ds(seg, _SEG)], y1_v)
    pltpu.sync_copy(c_hbm.at[img, pl.ds(seg, _SEG)], x2_v)
    pltpu.sync_copy(d_hbm.at[img, pl.ds(seg, _SEG)], y2_v)
    pltpu.sync_copy(ps_hbm.at[img, pl.ds(seg, _SEG)], sc_v)
    pltpu.sync_copy(nb_hbm, nb_v)

    lane = lax.iota(jnp.int32, 16)

    @plsc.parallel_loop(
        0, _SEG, step=16, unroll=8,
        carry=(jnp.full((16,), -2.0, jnp.float32),
               jnp.zeros((16,), jnp.int32)))
    def prep_carry(i, carry):
        rm, rc = carry
        s = pl.ds(i, 16)
        av = x1_v[s]
        bv = y1_v[s]
        cv = x2_v[s]
        dv = y2_v[s]
        x1 = jnp.minimum(av, cv) * W_IMG
        x2 = jnp.maximum(av, cv) * W_IMG
        y1 = jnp.minimum(bv, dv) * H_IMG
        y2 = jnp.maximum(bv, dv) * H_IMG
        bw = x2 - x1
        bh = y2 - y1
        colv = seg + i + lane
        m = (bh > _H_MIN) & (bw > _W_MIN) & (colv < NUM_INIT)
        sc = jnp.where(m, sc_v[s], -1.0)
        x1_v[s] = x1
        y1_v[s] = y1
        x2_v[s] = x2
        y2_v[s] = y2
        sc_v[s] = sc
        ar_v[s] = bw * bh
        upd = sc > rm
        return jnp.maximum(rm, sc), jnp.where(upd, i, rc)

    rm0, rc0 = prep_carry

    zf = jnp.zeros((16,), jnp.float32)
    for j in range(_SLOT // 16):
        s = pl.ds(j * 16, 16)
        ox1_v[s] = zf
        oy1_v[s] = zf
        ox2_v[s] = zf
        oy2_v[s] = zf

    nbv = nb_v[...].astype(jnp.float32)
    nf_f = jnp.sum(jnp.where(lane == img, nbv, 0.0))
    nf = jnp.clip(nf_f, np.float32(MIN_FINAL),
                  np.float32(MAX_FINAL - 1)).astype(jnp.int32)

    def round_cond(st):
        return st[0]

    def round_body(st):
        cont, r, k, act, rm, rc = st
        lmax = jnp.max(rm)
        lmax_v = jnp.full((16,), lmax, jnp.float32)
        gi_f = jnp.min(jnp.where(rm == lmax_v,
                                 (rc + lane + seg).astype(jnp.float32),
                                 _BIGF))
        gl = jnp.minimum(gi_f.astype(jnp.int32) - seg, _SEG - 1)
        gbase = lax.shift_left(lax.shift_right_logical(gl, 4), 4)
        gsel = lane == (gl & 15)
        gs = pl.ds(gbase, 16)
        x1l = jnp.full((16,), jnp.sum(jnp.where(gsel, x1_v[gs], 0.0)),
                       jnp.float32)
        y1l = jnp.full((16,), jnp.sum(jnp.where(gsel, y1_v[gs], 0.0)),
                       jnp.float32)
        x2l = jnp.full((16,), jnp.sum(jnp.where(gsel, x2_v[gs], 0.0)),
                       jnp.float32)
        y2l = jnp.full((16,), jnp.sum(jnp.where(gsel, y2_v[gs], 0.0)),
                       jnp.float32)
        cand_v[pl.ds(0, 16)] = lmax_v
        cand_v[pl.ds(16, 16)] = jnp.full((16,), gi_f, jnp.float32)
        cand_v[pl.ds(32, 16)] = x1l
        cand_v[pl.ds(48, 16)] = y1l
        cand_v[pl.ds(64, 16)] = x2l
        cand_v[pl.ds(80, 16)] = y2l
        cand_v[pl.ds(96, 16)] = jnp.where(act, jnp.full((16,), 1.0),
                                          jnp.full((16,), 0.0))

        par = r & 1
        woff = pl.multiple_of(par * 1792 + si * 112, 112)
        pltpu.sync_copy(cand_v, cand_sp.at[pl.ds(woff, 112)])
        plsc.subcore_barrier()
        roff = pl.multiple_of(par * 1792, 1792)
        pltpu.sync_copy(cand_sp.at[pl.ds(roff, 1792)], grp_v)

        gb = grp * 896
        sj = [grp_v[pl.ds(gb + j * 112, 16)] for j in range(8)]
        ij = [grp_v[pl.ds(gb + j * 112 + 16, 16)] for j in range(8)]
        wmax = sj[0]
        for j in range(1, 8):
            wmax = jnp.maximum(wmax, sj[j])
        widx = _BIGF * jnp.ones((16,), jnp.float32)
        for j in range(8):
            widx = jnp.minimum(widx, jnp.where(sj[j] == wmax, ij[j], _BIGF))
        selj = [(sj[j] == wmax) & (ij[j] == widx) for j in range(8)]

        def pick(off):
            acc = jnp.zeros((16,), jnp.float32)
            for j in range(8):
                acc = acc + jnp.where(selj[j],
                                      grp_v[pl.ds(gb + j * 112 + off, 16)],
                                      0.0)
            return acc

        x1m = pick(32)
        y1m = pick(48)
        x2m = pick(64)
        y2m = pick(80)
        am = (x2m - x1m) * (y2m - y1m)
        found = jnp.max(wmax) > -0.5

        @plsc.parallel_loop(
            0, _SEG, step=16, unroll=8,
            carry=(jnp.full((16,), -2.0, jnp.float32),
                   jnp.zeros((16,), jnp.int32)))
        def supp_carry(i, carry):
            rm2, rc2 = carry
            s = pl.ds(i, 16)
            x1 = x1_v[s]
            y1 = y1_v[s]
            x2 = x2_v[s]
            y2 = y2_v[s]
            ar = ar_v[s]
            sc = sc_v[s]
            xx1 = jnp.maximum(x1m, x1)
            yy1 = jnp.maximum(y1m, y1)
            xx2 = jnp.minimum(x2m, x2)
            yy2 = jnp.minimum(y2m, y2)
            w = jnp.maximum(0.0, xx2 - xx1)
            h = jnp.maximum(0.0, yy2 - yy1)
            inter = w * h
            iou = inter / (am + ar - inter + 1e-9)
            sc2 = jnp.where(iou > NMS_THR, -1.0, sc)
            sc_v[s] = sc2
            upd = sc2 > rm2
            return jnp.maximum(rm2, sc2), jnp.where(upd, i, rc2)

        rm3, rc3 = supp_carry

        write = found & (k < nf)

        @pl.when((slot == 0) & write)
        def _():
            kbase = lax.shift_left(lax.shift_right_logical(k, 4), 4)
            ks = pl.ds(kbase, 16)
            wsel = lane == (k & 15)
            ox1_v[ks] = jnp.where(wsel, x1m, ox1_v[ks])
            oy1_v[ks] = jnp.where(wsel, y1m, oy1_v[ks])
            ox2_v[ks] = jnp.where(wsel, x2m, ox2_v[ks])
            oy2_v[ks] = jnp.where(wsel, y2m, oy2_v[ks])

        k2 = k + jnp.where(write, 1, 0).astype(jnp.int32)
        act2 = write & (k2 < nf)

        sib = jnp.max(grp_v[pl.ds((1 - grp) * 896 + 96, 16)]) > 0.5
        cont2 = act | sib
        return cont2, r + 1, k2, act2, rm3, rc3

    _, _, k_fin, _, _, _ = lax.while_loop(
        round_cond, round_body,
        (jnp.bool_(True), jnp.int32(0), jnp.int32(0), jnp.bool_(True),
         rm0, rc0))

    @pl.when(slot == 0)
    def _():
        pltpu.sync_copy(ox1_v, ox1_hbm.at[img])
        pltpu.sync_copy(oy1_v, oy1_hbm.at[img])
        pltpu.sync_copy(ox2_v, ox2_hbm.at[img])
        pltpu.sync_copy(oy2_v, oy2_hbm.at[img])
        cnt_v[...] = jnp.full((16,), k_fin, jnp.int32)
        pltpu.sync_copy(cnt_v, cnt_hbm.at[img])


@functools.lru_cache(maxsize=1)
def _build_sc_kernel():
    mesh = plsc.VectorSubcoreMesh(core_axis_name="c", subcore_axis_name="s")
    f_out = jax.ShapeDtypeStruct((NUM_IMG, _SLOT), jnp.float32)
    i_out = jax.ShapeDtypeStruct((NUM_IMG, 16), jnp.int32)
    seg = pltpu.VMEM((_SEG,), jnp.float32)
    return pl.kernel(
        _sc_nms,
        out_type=(f_out, f_out, f_out, f_out, i_out),
        mesh=mesh,
        compiler_params=pltpu.CompilerParams(needs_layout_passes=False),
        scratch_types=[
            seg, seg, seg, seg, seg,
            pltpu.VMEM((16,), jnp.int32),
            pltpu.VMEM((_SLOT,), jnp.float32),
            pltpu.VMEM((_SLOT,), jnp.float32),
            pltpu.VMEM((_SLOT,), jnp.float32),
            pltpu.VMEM((_SLOT,), jnp.float32),
            pltpu.VMEM((16,), jnp.int32),
            pltpu.VMEM((112,), jnp.float32),
            pltpu.VMEM((1792,), jnp.float32),
            seg,
            pltpu.VMEM_SHARED((3584,), jnp.float32),
        ],
    )


def kernel(rand_boxes_init, pseudo_scores, num_of_boxes_per_img):
    pad = _NPAD - NUM_INIT
    a = jnp.pad(rand_boxes_init[..., 0], ((0, 0), (0, pad)))
    b = jnp.pad(rand_boxes_init[..., 1], ((0, 0), (0, pad)))
    c = jnp.pad(rand_boxes_init[..., 2], ((0, 0), (0, pad)))
    d = jnp.pad(rand_boxes_init[..., 3], ((0, 0), (0, pad)))
    ps = jnp.pad(pseudo_scores, ((0, 0), (0, pad)))
    nb = jnp.pad(num_of_boxes_per_img, (0, 16 - NUM_IMG))

    ox1, oy1, ox2, oy2, cnt = _build_sc_kernel()(a, b, c, d, ps, nb)

    out = jnp.stack([ox1[:, :MAX_FINAL], oy1[:, :MAX_FINAL],
                     ox2[:, :MAX_FINAL], oy2[:, :MAX_FINAL]], axis=-1)
    counts = cnt[:, 0]
    return out, counts
